# Initial kernel scaffold; baseline (speedup 1.0000x reference)
import jax
import jax.numpy as jnp
from jax.experimental import pallas as pl


def kernel(user_ids, question_ids, user_table, question_table, W, b):
    raise NotImplementedError("write your pallas kernel here")


# speedup vs baseline: 4.0554x; 4.0554x over previous
import jax
import jax.numpy as jnp
from jax import lax
from jax.experimental import pallas as pl

B = 16384
D = 32


def _dense_tc(u_emb, q_emb, W, b2d):
    BLK = 2048

    def body(u_ref, q_ref, w_ref, b_ref, o_ref):
        u = u_ref[...]
        q = q_ref[...]
        w = w_ref[...]
        bb = b_ref[...]
        logits = lax.dot_general(u, w[:, :D], (((1,), (1,)), ((), ())),
                                 preferred_element_type=jnp.float32)
        logits = logits + lax.dot_general(q, w[:, D:],
                                          (((1,), (1,)), ((), ())),
                                          preferred_element_type=jnp.float32)
        logits = logits + bb
        m = jnp.max(logits, axis=1, keepdims=True)
        e = jnp.exp(logits - m)
        o_ref[...] = e / jnp.sum(e, axis=1, keepdims=True)

    return pl.pallas_call(
        body,
        grid=(B // BLK,),
        in_specs=[
            pl.BlockSpec((BLK, D), lambda i: (i, 0)),
            pl.BlockSpec((BLK, D), lambda i: (i, 0)),
            pl.BlockSpec((2, 2 * D), lambda i: (0, 0)),
            pl.BlockSpec((1, 2), lambda i: (0, 0)),
        ],
        out_specs=pl.BlockSpec((BLK, 2), lambda i: (i, 0)),
        out_shape=jax.ShapeDtypeStruct((B, 2), jnp.float32),
    )(u_emb, q_emb, W, b2d)


def kernel(user_ids, question_ids, user_table, question_table, W, b):
    u_emb = jnp.take(user_table, user_ids, axis=0)
    q_emb = jnp.take(question_table, question_ids, axis=0)
    return _dense_tc(u_emb, q_emb, W, b.reshape(1, 2))


# speedup vs baseline: 5.7428x; 1.4161x over previous
import functools

import jax
import jax.numpy as jnp
from jax import lax
from jax.experimental import pallas as pl
from jax.experimental.pallas import tpu as pltpu
from jax.experimental.pallas import tpu_sc as plsc

B = 16384
D = 32
NU = 1000000
NQ = 100000

_info = plsc.get_sparse_core_info()
_NC, _NS = _info.num_cores, _info.num_subcores
_NW = _NC * _NS
_BPW = B // _NW
_CHUNK = 128
_NCH = _BPW // _CHUNK


def _project_tc(tab_t, W, b2d, first_half, add_bias, blk):
    n = tab_t.shape[1]
    grid = (pl.cdiv(n, blk),)

    def body(t_ref, w_ref, b_ref, o_ref):
        w = w_ref[...]
        if first_half:
            wd = w[0:1, :D] - w[1:2, :D]
        else:
            wd = w[0:1, D:] - w[1:2, D:]
        s = lax.dot_general(wd, t_ref[...], (((1,), (0,)), ((), ())),
                            preferred_element_type=jnp.float32)
        s = s[0]
        if add_bias:
            bb = b_ref[...]
            s = s + (bb[0, 0] - bb[0, 1])
        o_ref[...] = s

    return pl.pallas_call(
        body,
        grid=grid,
        in_specs=[
            pl.BlockSpec((D, blk), lambda i: (0, i)),
            pl.BlockSpec((2, 2 * D), lambda i: (0, 0)),
            pl.BlockSpec((1, 2), lambda i: (0, 0)),
        ],
        out_specs=pl.BlockSpec((blk,), lambda i: (i,)),
        out_shape=jax.ShapeDtypeStruct((n,), jnp.float32),
    )(tab_t, W, b2d)


def _gather_sigmoid_sc(s_u, s_q, user_ids, question_ids):
    mesh = plsc.VectorSubcoreMesh(core_axis_name="c", subcore_axis_name="s")

    @functools.partial(
        pl.kernel,
        mesh=mesh,
        out_type=[
            jax.ShapeDtypeStruct((B,), jnp.float32),
            jax.ShapeDtypeStruct((B,), jnp.float32),
        ],
        scratch_types=[
            pltpu.VMEM((_NCH, _CHUNK), jnp.int32),
            pltpu.VMEM((_NCH, _CHUNK), jnp.int32),
            pltpu.VMEM((_BPW,), jnp.float32),
            pltpu.VMEM((_BPW,), jnp.float32),
            pltpu.VMEM((_BPW,), jnp.float32),
            pltpu.VMEM((_BPW,), jnp.float32),
            pltpu.SemaphoreType.DMA,
        ],
    )
    def body(su_hbm, sq_hbm, uids, qids, p0_hbm, p1_hbm,
             uidx, qidx, su_v, sq_v, p0_v, p1_v, sem):
        wid = lax.axis_index("s") * _NC + lax.axis_index("c")
        base = wid * _BPW
        for j in range(_NCH):
            pltpu.sync_copy(uids.at[pl.ds(base + j * _CHUNK, _CHUNK)],
                            uidx.at[j])
            pltpu.sync_copy(qids.at[pl.ds(base + j * _CHUNK, _CHUNK)],
                            qidx.at[j])
        copies = []
        for j in range(_NCH):
            copies.append(pltpu.async_copy(
                su_hbm.at[uidx.at[j]],
                su_v.at[pl.ds(j * _CHUNK, _CHUNK)], sem))
            copies.append(pltpu.async_copy(
                sq_hbm.at[qidx.at[j]],
                sq_v.at[pl.ds(j * _CHUNK, _CHUNK)], sem))
        for c in copies:
            c.wait()
        for k in range(_BPW // 16):
            sl = pl.ds(k * 16, 16)
            ld = su_v[sl] + sq_v[sl]
            p0 = 1.0 / (1.0 + jnp.exp(-ld))
            p0_v[sl] = p0
            p1_v[sl] = 1.0 - p0
        pltpu.sync_copy(p0_v, p0_hbm.at[pl.ds(base, _BPW)])
        pltpu.sync_copy(p1_v, p1_hbm.at[pl.ds(base, _BPW)])

    return body(s_u, s_q, user_ids, question_ids)


def kernel(user_ids, question_ids, user_table, question_table, W, b):
    b2d = b.reshape(1, 2)
    s_u = _project_tc(user_table.T, W, b2d, first_half=True,
                      add_bias=True, blk=65536)
    s_q = _project_tc(question_table.T, W, b2d, first_half=False,
                      add_bias=False, blk=65536)
    p0, p1 = _gather_sigmoid_sc(s_u, s_q, user_ids.astype(jnp.int32),
                                question_ids.astype(jnp.int32))
    return jnp.stack([p0, p1], axis=-1)


# speedup vs baseline: 6.0122x; 1.0469x over previous
import functools

import jax
import jax.numpy as jnp
from jax import lax
from jax.experimental import pallas as pl
from jax.experimental.pallas import tpu as pltpu
from jax.experimental.pallas import tpu_sc as plsc

B = 16384
D = 32
NU = 1000000
NQ = 100000

_info = plsc.get_sparse_core_info()
_NC, _NS = _info.num_cores, _info.num_subcores
_NW = _NC * _NS
_BPW = B // _NW
_CHUNK = 128
_NCH = _BPW // _CHUNK


def _project_tc(tab_t, W, b2d, first_half, add_bias, blk):
    n = tab_t.shape[1]
    grid = (pl.cdiv(n, blk),)

    def body(t_ref, w_ref, b_ref, o_ref):
        w = w_ref[...]
        if first_half:
            wd = w[0:1, :D] - w[1:2, :D]
        else:
            wd = w[0:1, D:] - w[1:2, D:]
        s = lax.dot_general(wd, t_ref[...], (((1,), (0,)), ((), ())),
                            preferred_element_type=jnp.float32)
        s = s[0]
        if add_bias:
            bb = b_ref[...]
            s = s + (bb[0, 0] - bb[0, 1])
        o_ref[...] = s

    return pl.pallas_call(
        body,
        grid=grid,
        in_specs=[
            pl.BlockSpec((D, blk), lambda i: (0, i)),
            pl.BlockSpec((2, 2 * D), lambda i: (0, 0)),
            pl.BlockSpec((1, 2), lambda i: (0, 0)),
        ],
        out_specs=pl.BlockSpec((blk,), lambda i: (i,)),
        out_shape=jax.ShapeDtypeStruct((n,), jnp.float32),
    )(tab_t, W, b2d)


def _gather_sigmoid_sc(s_u, s_q, user_ids, question_ids):
    mesh = plsc.VectorSubcoreMesh(core_axis_name="c", subcore_axis_name="s")

    @functools.partial(
        pl.kernel,
        mesh=mesh,
        out_type=[
            jax.ShapeDtypeStruct((B,), jnp.float32),
            jax.ShapeDtypeStruct((B,), jnp.float32),
        ],
        scratch_types=[
            pltpu.VMEM((_NCH, _CHUNK), jnp.int32),
            pltpu.VMEM((_NCH, _CHUNK), jnp.int32),
            pltpu.VMEM((_BPW,), jnp.float32),
            pltpu.VMEM((_BPW,), jnp.float32),
            pltpu.VMEM((_BPW,), jnp.float32),
            pltpu.VMEM((_BPW,), jnp.float32),
            pltpu.SemaphoreType.DMA,
        ],
    )
    def body(su_hbm, sq_hbm, uids, qids, p0_hbm, p1_hbm,
             uidx, qidx, su_v, sq_v, p0_v, p1_v, sem):
        wid = lax.axis_index("s") * _NC + lax.axis_index("c")
        base = wid * _BPW
        cu = pltpu.async_copy(uids.at[pl.ds(wid * _NCH, _NCH)], uidx, sem)
        cq = pltpu.async_copy(qids.at[pl.ds(wid * _NCH, _NCH)], qidx, sem)
        cu.wait()
        cq.wait()
        copies = []
        for j in range(_NCH):
            copies.append(pltpu.async_copy(
                su_hbm.at[uidx.at[j]],
                su_v.at[pl.ds(j * _CHUNK, _CHUNK)], sem))
            copies.append(pltpu.async_copy(
                sq_hbm.at[qidx.at[j]],
                sq_v.at[pl.ds(j * _CHUNK, _CHUNK)], sem))
        for c in copies:
            c.wait()
        for k in range(_BPW // 16):
            sl = pl.ds(k * 16, 16)
            ld = su_v[sl] + sq_v[sl]
            p0 = 1.0 / (1.0 + jnp.exp(-ld))
            p0_v[sl] = p0
            p1_v[sl] = 1.0 - p0
        pltpu.sync_copy(p0_v, p0_hbm.at[pl.ds(base, _BPW)])
        pltpu.sync_copy(p1_v, p1_hbm.at[pl.ds(base, _BPW)])

    return body(s_u, s_q, user_ids, question_ids)


def kernel(user_ids, question_ids, user_table, question_table, W, b):
    b2d = b.reshape(1, 2)
    s_u = _project_tc(user_table.T, W, b2d, first_half=True,
                      add_bias=True, blk=65536)
    s_q = _project_tc(question_table.T, W, b2d, first_half=False,
                      add_bias=False, blk=65536)
    uids2d = user_ids.astype(jnp.int32).reshape(B // _CHUNK, _CHUNK)
    qids2d = question_ids.astype(jnp.int32).reshape(B // _CHUNK, _CHUNK)
    p0, p1 = _gather_sigmoid_sc(s_u, s_q, uids2d, qids2d)
    return jnp.stack([p0, p1], axis=-1)
